# baseline (speedup 1.0000x reference)
import functools

import jax
import jax.numpy as jnp
from jax import lax
from jax.experimental import pallas as pl
from jax.experimental.pallas import tpu as pltpu
from jax.experimental.pallas import tpu_sc as plsc

N = 16384
C = 1000

_INFO = plsc.get_sparse_core_info()
NC, NS, L = _INFO.num_cores, _INFO.num_subcores, _INFO.num_lanes
NW = NC * NS
RPW = N // NW
CHUNK = 64
NCHUNK = RPW // CHUNK
GROUPS = CHUNK // L

_mesh = plsc.VectorSubcoreMesh(core_axis_name="c", subcore_axis_name="s")


@functools.partial(
    pl.kernel,
    out_type=jax.ShapeDtypeStruct((N * C,), jnp.float32),
    mesh=_mesh,
    scratch_types=[
        pltpu.VMEM((RPW,), jnp.int32),
        pltpu.VMEM((CHUNK * C,), jnp.float32),
        pltpu.VMEM((CHUNK * C,), jnp.float32),
        pltpu.SemaphoreType.DMA,
        pltpu.SemaphoreType.DMA,
    ],
    compiler_params=pltpu.CompilerParams(needs_layout_passes=False),
)
def _one_hot_sc(x_hbm, zeros_hbm, out_hbm, idx_v, buf0, buf1, sem0, sem1):
    wid = lax.axis_index("s") * NC + lax.axis_index("c")
    wbase = wid * RPW

    pltpu.sync_copy(x_hbm.at[pl.ds(wbase, RPW)], idx_v)
    pltpu.sync_copy(zeros_hbm, buf0)
    pltpu.sync_copy(zeros_hbm, buf1)

    bufs = (buf0, buf1)
    sems = (sem0, sem1)
    lane = lax.iota(jnp.int32, L)
    ones = jnp.full((L,), 1.0, jnp.float32)
    zeros = jnp.zeros((L,), jnp.float32)

    dmas = []
    for c in range(NCHUNK):
        buf = bufs[c % 2]
        sem = sems[c % 2]
        if c >= 2:
            dmas[c - 2].wait()
            for g in range(GROUPS):
                pos = (lane + g * L) * C + idx_v[
                    pl.ds((c - 2) * CHUNK + g * L, L)
                ]
                plsc.store_scatter(buf, [pos], zeros)
        for g in range(GROUPS):
            pos = (lane + g * L) * C + idx_v[pl.ds(c * CHUNK + g * L, L)]
            plsc.store_scatter(buf, [pos], ones)
        dmas.append(
            pltpu.async_copy(
                buf,
                out_hbm.at[pl.ds((wbase + c * CHUNK) * C, CHUNK * C)],
                sem,
            )
        )
    dmas[NCHUNK - 2].wait()
    dmas[NCHUNK - 1].wait()


def kernel(x1):
    x = x1.astype(jnp.int32)
    zeros = jnp.zeros((CHUNK * C,), jnp.float32)
    return _one_hot_sc(x, zeros).reshape(N, C)


# speedup vs baseline: 1.5407x; 1.5407x over previous
import functools

import jax
import jax.numpy as jnp
from jax import lax
from jax.experimental import pallas as pl
from jax.experimental.pallas import tpu as pltpu
from jax.experimental.pallas import tpu_sc as plsc

N = 16384
C = 1000

_INFO = plsc.get_sparse_core_info()
NC, NS, L = _INFO.num_cores, _INFO.num_subcores, _INFO.num_lanes
NW = NC * NS
RPW = N // NW
CHUNK = 32
NCHUNK = RPW // CHUNK
GROUPS = CHUNK // L

_mesh = plsc.VectorSubcoreMesh(core_axis_name="c", subcore_axis_name="s")


@functools.partial(
    pl.kernel,
    out_type=jax.ShapeDtypeStruct((N, C), jnp.float32),
    mesh=_mesh,
    scratch_types=[
        pltpu.VMEM((RPW,), jnp.int32),
        pltpu.VMEM((CHUNK, C), jnp.float32),
        pltpu.VMEM((CHUNK, C), jnp.float32),
        pltpu.SemaphoreType.DMA,
        pltpu.SemaphoreType.DMA,
    ],
    compiler_params=pltpu.CompilerParams(needs_layout_passes=False),
)
def _one_hot_sc(x_hbm, zeros_hbm, out_hbm, idx_v, buf0, buf1, sem0, sem1):
    wid = lax.axis_index("s") * NC + lax.axis_index("c")
    wbase = wid * RPW

    pltpu.sync_copy(x_hbm.at[pl.ds(wbase, RPW)], idx_v)
    pltpu.sync_copy(zeros_hbm, buf0)
    pltpu.sync_copy(zeros_hbm, buf1)

    bufs = (buf0, buf1)
    sems = (sem0, sem1)
    lane = lax.iota(jnp.int32, L)
    ones = jnp.full((L,), 1.0, jnp.float32)
    zeros = jnp.zeros((L,), jnp.float32)

    dmas = []
    for c in range(NCHUNK):
        buf = bufs[c % 2]
        sem = sems[c % 2]
        if c >= 2:
            dmas[c - 2].wait()
            for g in range(GROUPS):
                rows = lane + g * L
                cols = idx_v[pl.ds((c - 2) * CHUNK + g * L, L)]
                plsc.store_scatter(buf, [rows, cols], zeros)
        for g in range(GROUPS):
            rows = lane + g * L
            cols = idx_v[pl.ds(c * CHUNK + g * L, L)]
            plsc.store_scatter(buf, [rows, cols], ones)
        dmas.append(
            pltpu.async_copy(
                buf,
                out_hbm.at[pl.ds(wbase + c * CHUNK, CHUNK)],
                sem,
            )
        )
    dmas[NCHUNK - 2].wait()
    dmas[NCHUNK - 1].wait()


def kernel(x1):
    x = x1.astype(jnp.int32)
    zeros = jnp.zeros((CHUNK, C), jnp.float32)
    return _one_hot_sc(x, zeros)


# speedup vs baseline: 3.2129x; 2.0853x over previous
import functools

import jax
import jax.numpy as jnp
from jax import lax
from jax.experimental import pallas as pl
from jax.experimental.pallas import tpu as pltpu
from jax.experimental.pallas import tpu_sc as plsc

N = 16384
C = 1000

_INFO = plsc.get_sparse_core_info()
NC, NS, L = _INFO.num_cores, _INFO.num_subcores, _INFO.num_lanes
NW = NC * NS
CPW = N // NW
CB = 128
NBLK = CPW // CB
GROUPS = CB // L

_mesh = plsc.VectorSubcoreMesh(core_axis_name="c", subcore_axis_name="s")


@functools.partial(
    pl.kernel,
    out_type=jax.ShapeDtypeStruct((C, N), jnp.float32),
    mesh=_mesh,
    scratch_types=[
        pltpu.VMEM((CB,), jnp.int32),
        pltpu.VMEM((C, CB), jnp.float32),
        pltpu.SemaphoreType.DMA,
    ],
    compiler_params=pltpu.CompilerParams(needs_layout_passes=False),
)
def _one_hot_t_sc(x_hbm, zeros_hbm, out_hbm, idx_v, buf, sem):
    wid = lax.axis_index("s") * NC + lax.axis_index("c")
    wbase = wid * CPW

    pltpu.sync_copy(zeros_hbm, buf)

    lane = lax.iota(jnp.int32, L)
    ones = jnp.full((L,), 1.0, jnp.float32)
    zeros = jnp.zeros((L,), jnp.float32)

    d = None
    for c in range(NBLK):
        if d is not None:
            d.wait()
            for g in range(GROUPS):
                cols = lane + g * L
                cls = idx_v[pl.ds(g * L, L)]
                plsc.store_scatter(buf, [cls, cols], zeros)
        pltpu.sync_copy(x_hbm.at[pl.ds(wbase + c * CB, CB)], idx_v)
        for g in range(GROUPS):
            cols = lane + g * L
            cls = idx_v[pl.ds(g * L, L)]
            plsc.store_scatter(buf, [cls, cols], ones)
        d = pltpu.async_copy(
            buf, out_hbm.at[:, pl.ds(wbase + c * CB, CB)], sem
        )
    d.wait()


def kernel(x1):
    x = x1.astype(jnp.int32)
    zeros = jnp.zeros((C, CB), jnp.float32)
    return _one_hot_t_sc(x, zeros).T
